# baseline (speedup 1.0000x reference)
import functools

import jax
import jax.numpy as jnp
from jax.experimental import pallas as pl

B = 8
N_SENT = 16
SEQ = 64
D_EM = 256
D_HID = 256
BN = B * N_SENT
TC = 8
HB = BN // 2


def _gru_body(x_ref, lens_ref, wih_ref, whh_ref, bih_ref, bhh_ref,
              out_ref):
    lens = lens_ref[...]
    wih = wih_ref[...]
    whh = whh_ref[...]
    bih = bih_ref[...]
    bhh = bhh_ref[...]
    la = lens[:HB, :]
    lb = lens[HB:, :]

    def step(h, gi, lens_h, t):
        gh = jnp.dot(h, whh, preferred_element_type=jnp.float32) + bhh
        r = 0.5 + 0.5 * jnp.tanh(0.5 * (gi[:, :D_HID] + gh[:, :D_HID]))
        z = 0.5 + 0.5 * jnp.tanh(
            0.5 * (gi[:, D_HID:2 * D_HID] + gh[:, D_HID:2 * D_HID]))
        n = jnp.tanh(gi[:, 2 * D_HID:] + r * gh[:, 2 * D_HID:])
        h_new = n + z * (h - n)
        valid = t < lens_h
        return jnp.where(valid, h_new, h)

    ha = jnp.zeros((HB, D_HID), dtype=jnp.float32)
    hb = jnp.zeros((HB, D_HID), dtype=jnp.float32)
    for c in range(SEQ // TC):
        gis = [
            jnp.dot(x_ref[:, c * TC + t, :], wih,
                    preferred_element_type=jnp.float32) + bih
            for t in range(TC)
        ]
        for t in range(TC):
            ha = step(ha, gis[t][:HB, :], la, c * TC + t)
            hb = step(hb, gis[t][HB:, :], lb, c * TC + t)
    out_ref[:HB, :] = ha
    out_ref[HB:, :] = hb


@functools.partial(jax.jit, static_argnames=())
def kernel(inputs, mask, W_ih, W_hh, b_ih, b_hh):
    x = inputs.reshape(BN, SEQ, D_EM)
    lens = mask.reshape(BN, SEQ).sum(axis=1, keepdims=True)
    bih = b_ih.reshape(1, 3 * D_HID)
    bhh = b_hh.reshape(1, 3 * D_HID)

    h_final = pl.pallas_call(
        _gru_body,
        out_shape=jax.ShapeDtypeStruct((BN, D_HID), jnp.float32),
    )(x, lens, W_ih, W_hh, bih, bhh)

    return h_final.reshape(B, N_SENT, D_HID)
